# Initial kernel scaffold; baseline (speedup 1.0000x reference)
import jax
import jax.numpy as jnp
from jax.experimental import pallas as pl


def kernel(lm_X, lm_Y, tg_X, tg_Y, lm_delay, tg_delay, emb_W, emb_b, w1, w2):
    raise NotImplementedError("write your pallas kernel here")


# speedup vs baseline: 2.8700x; 2.8700x over previous
import functools

import jax
import jax.numpy as jnp
from jax.experimental import pallas as pl

_N1 = 1024
_N2 = 1024
_DIM = 32
_EMB = 64
_HEADS = 4
_K = 5
_N = _N1 + _N2

_HP = jax.lax.Precision.HIGHEST


def _prologue_body(feats_ref, w_ref, b_ref, w1t_ref, w2t_ref, a_ref, bb_ref):
    emb = jax.lax.dot(feats_ref[...], w_ref[...], precision=_HP) + b_ref[...]
    for h in range(_HEADS):
        ah = emb * w1t_ref[h:h + 1, :]
        bh = emb * w2t_ref[h:h + 1, :]
        na = jnp.sqrt(jnp.sum(ah * ah, axis=1, keepdims=True))
        nb = jnp.sqrt(jnp.sum(bh * bh, axis=1, keepdims=True))
        a_ref[:, h * _EMB:(h + 1) * _EMB] = ah / jnp.maximum(na, 1e-20)
        bb_ref[:, h * _EMB:(h + 1) * _EMB] = bh / jnp.maximum(nb, 1e-20)


def _d2_body(xb_ref, xt_ref, out_ref):
    i = pl.program_id(0)
    rows = xb_ref.shape[0]
    acc = jnp.zeros((rows, _N1), jnp.float32)
    for d in range(_DIM):
        diff = xb_ref[:, d:d + 1] - xt_ref[d:d + 1, :]
        acc = acc + diff * diff
    col = jax.lax.broadcasted_iota(jnp.int32, (rows, _N1), 1)
    row = jax.lax.broadcasted_iota(jnp.int32, (rows, _N1), 0) + i * rows
    out_ref[...] = acc + jnp.where(col == row, 1e9, 0.0).astype(jnp.float32)


def _knn_body(d2_ref, y_ref, yp_ref, t_ref):
    i = pl.program_id(0)
    rows = t_ref.shape[0]

    @pl.when(i < _N1 // rows)
    def _top():
        work = d2_ref[...]
        colk = jax.lax.broadcasted_iota(jnp.int32, (rows, _N1), 1)
        acc = jnp.zeros((rows, _N1), jnp.float32)
        for _ in range(_K):
            m = jnp.min(work, axis=1, keepdims=True)
            eq = work == m
            idx = jnp.min(jnp.where(eq, colk, jnp.int32(2**30)), axis=1,
                          keepdims=True)
            oh = colk == idx
            acc = acc + oh.astype(jnp.float32)
            work = jnp.where(oh, jnp.float32(jnp.inf), work)
        t_ref[:, :_N1] = acc
        t_ref[:, _N1:] = jnp.ones((rows, _N2), jnp.float32)
        yp_ref[...] = jax.lax.dot(acc, y_ref[...], precision=_HP) * (1.0 / _K)

    @pl.when(i >= _N1 // rows)
    def _const():
        col = jax.lax.broadcasted_iota(jnp.int32, (rows, _N), 1)
        t_ref[...] = jnp.where(col < _N1, 1.0, 0.0).astype(jnp.float32)


def _adj_body(a_ref, b_ref, out_ref):
    rows = a_ref.shape[0]
    acc = jnp.zeros((rows, _N), jnp.float32)
    for h in range(_HEADS):
        ah = a_ref[:, h * _EMB:(h + 1) * _EMB]
        bh = b_ref[:, h * _EMB:(h + 1) * _EMB]
        dots = jax.lax.dot_general(ah, bh, (((1,), (1,)), ((), ())),
                                   precision=_HP)
        acc = acc + jax.nn.sigmoid(dots)
    out_ref[...] = acc * (1.0 / _HEADS)


def kernel(lm_X, lm_Y, tg_X, tg_Y, lm_delay, tg_delay, emb_W, emb_b, w1, w2):
    feats = jnp.concatenate([
        jnp.concatenate([lm_X, lm_delay[:, None]], axis=1),
        jnp.concatenate([tg_X, tg_delay[:, None]], axis=1),
    ], axis=0)

    pr_rows = 256
    ahat, bhat = pl.pallas_call(
        _prologue_body,
        grid=(_N // pr_rows,),
        in_specs=[
            pl.BlockSpec((pr_rows, _DIM + 1), lambda i: (i, 0)),
            pl.BlockSpec((_DIM + 1, _EMB), lambda i: (0, 0)),
            pl.BlockSpec((1, _EMB), lambda i: (0, 0)),
            pl.BlockSpec((_HEADS, _EMB), lambda i: (0, 0)),
            pl.BlockSpec((_HEADS, _EMB), lambda i: (0, 0)),
        ],
        out_specs=[
            pl.BlockSpec((pr_rows, _HEADS * _EMB), lambda i: (i, 0)),
            pl.BlockSpec((pr_rows, _HEADS * _EMB), lambda i: (i, 0)),
        ],
        out_shape=[
            jax.ShapeDtypeStruct((_N, _HEADS * _EMB), jnp.float32),
            jax.ShapeDtypeStruct((_N, _HEADS * _EMB), jnp.float32),
        ],
    )(feats, emb_W, emb_b.reshape(1, _EMB), w1.T, w2.T)

    d2_rows = 128
    d2 = pl.pallas_call(
        _d2_body,
        grid=(_N1 // d2_rows,),
        in_specs=[
            pl.BlockSpec((d2_rows, _DIM), lambda i: (i, 0)),
            pl.BlockSpec((_DIM, _N1), lambda i: (0, 0)),
        ],
        out_specs=pl.BlockSpec((d2_rows, _N1), lambda i: (i, 0)),
        out_shape=jax.ShapeDtypeStruct((_N1, _N1), jnp.float32),
    )(lm_X, lm_X.T)

    k_rows = 128
    nb_lm = _N1 // k_rows
    y_pred, teacher = pl.pallas_call(
        _knn_body,
        grid=(_N // k_rows,),
        in_specs=[
            pl.BlockSpec((k_rows, _N1), lambda i: (jnp.minimum(i, nb_lm - 1), 0)),
            pl.BlockSpec((_N1, 2), lambda i: (0, 0)),
        ],
        out_specs=[
            pl.BlockSpec((k_rows, 2), lambda i: (jnp.minimum(i, nb_lm - 1), 0)),
            pl.BlockSpec((k_rows, _N), lambda i: (i, 0)),
        ],
        out_shape=[
            jax.ShapeDtypeStruct((_N1, 2), jnp.float32),
            jax.ShapeDtypeStruct((_N, _N), jnp.float32),
        ],
    )(d2, lm_Y)

    a_rows = 128
    adj = pl.pallas_call(
        _adj_body,
        grid=(_N // a_rows,),
        in_specs=[
            pl.BlockSpec((a_rows, _HEADS * _EMB), lambda i: (i, 0)),
            pl.BlockSpec((_N, _HEADS * _EMB), lambda i: (0, 0)),
        ],
        out_specs=pl.BlockSpec((a_rows, _N), lambda i: (i, 0)),
        out_shape=jax.ShapeDtypeStruct((_N, _N), jnp.float32),
    )(ahat, bhat)

    return y_pred, adj, teacher


# speedup vs baseline: 4.0778x; 1.4208x over previous
import functools

import jax
import jax.numpy as jnp
from jax.experimental import pallas as pl

_N1 = 1024
_N2 = 1024
_DIM = 32
_EMB = 64
_HEADS = 4
_K = 5
_N = _N1 + _N2

_HP = jax.lax.Precision.HIGHEST


def _prologue_body(feats_ref, w_ref, b_ref, w1t_ref, w2t_ref, a_ref, bb_ref):
    emb = jax.lax.dot(feats_ref[...], w_ref[...], precision=_HP) + b_ref[...]
    for h in range(_HEADS):
        ah = emb * w1t_ref[h:h + 1, :]
        bh = emb * w2t_ref[h:h + 1, :]
        na = jnp.sqrt(jnp.sum(ah * ah, axis=1, keepdims=True))
        nb = jnp.sqrt(jnp.sum(bh * bh, axis=1, keepdims=True))
        a_ref[:, h * _EMB:(h + 1) * _EMB] = (
            ah / jnp.maximum(na, 1e-20)).astype(jnp.bfloat16)
        bb_ref[:, h * _EMB:(h + 1) * _EMB] = (
            bh / jnp.maximum(nb, 1e-20)).astype(jnp.bfloat16)


def _d2_body(xb_ref, xt_ref, out_ref):
    i = pl.program_id(0)
    rows = xb_ref.shape[0]
    acc = jnp.zeros((rows, _N1), jnp.float32)
    for d in range(_DIM):
        diff = xb_ref[:, d:d + 1] - xt_ref[d:d + 1, :]
        acc = acc + diff * diff
    col = jax.lax.broadcasted_iota(jnp.int32, (rows, _N1), 1)
    row = jax.lax.broadcasted_iota(jnp.int32, (rows, _N1), 0) + i * rows
    out_ref[...] = acc + jnp.where(col == row, 1e9, 0.0).astype(jnp.float32)


def _knn_body(d2_ref, y_ref, yp_ref, t_ref):
    i = pl.program_id(0)
    rows = t_ref.shape[0]

    @pl.when(i < _N1 // rows)
    def _top():
        work = d2_ref[...]
        colk = jax.lax.broadcasted_iota(jnp.int32, (rows, _N1), 1)
        acc = jnp.zeros((rows, _N1), jnp.float32)
        for _ in range(_K):
            m = jnp.min(work, axis=1, keepdims=True)
            eq = work == m
            idx = jnp.min(jnp.where(eq, colk, jnp.int32(2**30)), axis=1,
                          keepdims=True)
            oh = colk == idx
            acc = acc + oh.astype(jnp.float32)
            work = jnp.where(oh, jnp.float32(jnp.inf), work)
        t_ref[:, :_N1] = acc
        t_ref[:, _N1:] = jnp.ones((rows, _N2), jnp.float32)
        yp_ref[...] = jax.lax.dot(acc, y_ref[...], precision=_HP) * (1.0 / _K)

    @pl.when(i >= _N1 // rows)
    def _const():
        col = jax.lax.broadcasted_iota(jnp.int32, (rows, _N), 1)
        t_ref[...] = jnp.where(col < _N1, 1.0, 0.0).astype(jnp.float32)


def _adj_body(a_ref, b_ref, out_ref):
    rows = a_ref.shape[0]
    acc = jnp.zeros((rows, _N), jnp.float32)
    for h in range(_HEADS):
        ah = a_ref[:, h * _EMB:(h + 1) * _EMB]
        bh = b_ref[:, h * _EMB:(h + 1) * _EMB]
        dots = jax.lax.dot_general(ah, bh, (((1,), (1,)), ((), ())),
                                   preferred_element_type=jnp.float32)
        acc = acc + jax.nn.sigmoid(dots)
    out_ref[...] = acc * (1.0 / _HEADS)


def kernel(lm_X, lm_Y, tg_X, tg_Y, lm_delay, tg_delay, emb_W, emb_b, w1, w2):
    feats = jnp.concatenate([
        jnp.concatenate([lm_X, lm_delay[:, None]], axis=1),
        jnp.concatenate([tg_X, tg_delay[:, None]], axis=1),
    ], axis=0)

    pr_rows = 256
    ahat, bhat = pl.pallas_call(
        _prologue_body,
        grid=(_N // pr_rows,),
        in_specs=[
            pl.BlockSpec((pr_rows, _DIM + 1), lambda i: (i, 0)),
            pl.BlockSpec((_DIM + 1, _EMB), lambda i: (0, 0)),
            pl.BlockSpec((1, _EMB), lambda i: (0, 0)),
            pl.BlockSpec((_HEADS, _EMB), lambda i: (0, 0)),
            pl.BlockSpec((_HEADS, _EMB), lambda i: (0, 0)),
        ],
        out_specs=[
            pl.BlockSpec((pr_rows, _HEADS * _EMB), lambda i: (i, 0)),
            pl.BlockSpec((pr_rows, _HEADS * _EMB), lambda i: (i, 0)),
        ],
        out_shape=[
            jax.ShapeDtypeStruct((_N, _HEADS * _EMB), jnp.bfloat16),
            jax.ShapeDtypeStruct((_N, _HEADS * _EMB), jnp.bfloat16),
        ],
    )(feats, emb_W, emb_b.reshape(1, _EMB), w1.T, w2.T)

    d2_rows = 128
    d2 = pl.pallas_call(
        _d2_body,
        grid=(_N1 // d2_rows,),
        in_specs=[
            pl.BlockSpec((d2_rows, _DIM), lambda i: (i, 0)),
            pl.BlockSpec((_DIM, _N1), lambda i: (0, 0)),
        ],
        out_specs=pl.BlockSpec((d2_rows, _N1), lambda i: (i, 0)),
        out_shape=jax.ShapeDtypeStruct((_N1, _N1), jnp.float32),
    )(lm_X, lm_X.T)

    k_rows = 128
    nb_lm = _N1 // k_rows
    y_pred, teacher = pl.pallas_call(
        _knn_body,
        grid=(_N // k_rows,),
        in_specs=[
            pl.BlockSpec((k_rows, _N1), lambda i: (jnp.minimum(i, nb_lm - 1), 0)),
            pl.BlockSpec((_N1, 2), lambda i: (0, 0)),
        ],
        out_specs=[
            pl.BlockSpec((k_rows, 2), lambda i: (jnp.minimum(i, nb_lm - 1), 0)),
            pl.BlockSpec((k_rows, _N), lambda i: (i, 0)),
        ],
        out_shape=[
            jax.ShapeDtypeStruct((_N1, 2), jnp.float32),
            jax.ShapeDtypeStruct((_N, _N), jnp.float32),
        ],
    )(d2, lm_Y)

    a_rows = 128
    adj = pl.pallas_call(
        _adj_body,
        grid=(_N // a_rows,),
        in_specs=[
            pl.BlockSpec((a_rows, _HEADS * _EMB), lambda i: (i, 0)),
            pl.BlockSpec((_N, _HEADS * _EMB), lambda i: (0, 0)),
        ],
        out_specs=pl.BlockSpec((a_rows, _N), lambda i: (i, 0)),
        out_shape=jax.ShapeDtypeStruct((_N, _N), jnp.float32),
    )(ahat, bhat)

    return y_pred, adj, teacher


# speedup vs baseline: 4.1526x; 1.0184x over previous
import functools

import jax
import jax.numpy as jnp
from jax.experimental import pallas as pl

_N1 = 1024
_N2 = 1024
_DIM = 32
_EMB = 64
_HEADS = 4
_K = 5
_N = _N1 + _N2

_HP = jax.lax.Precision.HIGHEST


def _prologue_body(feats_ref, w_ref, b_ref, w1t_ref, w2t_ref, a_ref, bb_ref):
    emb = jax.lax.dot(feats_ref[...], w_ref[...], precision=_HP) + b_ref[...]
    for h in range(_HEADS):
        ah = emb * w1t_ref[h:h + 1, :]
        bh = emb * w2t_ref[h:h + 1, :]
        na = jnp.sqrt(jnp.sum(ah * ah, axis=1, keepdims=True))
        nb = jnp.sqrt(jnp.sum(bh * bh, axis=1, keepdims=True))
        a_ref[:, h * _EMB:(h + 1) * _EMB] = (
            ah / jnp.maximum(na, 1e-20)).astype(jnp.bfloat16)
        bb_ref[:, h * _EMB:(h + 1) * _EMB] = (
            bh / jnp.maximum(nb, 1e-20)).astype(jnp.bfloat16)


def _knn_body(xb_ref, xt_ref, y_ref, yp_ref, t_ref):
    i = pl.program_id(0)
    rows = yp_ref.shape[0]

    @pl.when(i < _N1 // rows)
    def _top():
        work = jnp.zeros((rows, _N1), jnp.float32)
        for d in range(_DIM):
            diff = xb_ref[:, d:d + 1] - xt_ref[d:d + 1, :]
            work = work + diff * diff
        colk = jax.lax.broadcasted_iota(jnp.int32, (rows, _N1), 1)
        rowk = jax.lax.broadcasted_iota(jnp.int32, (rows, _N1), 0) + i * rows
        work = work + jnp.where(colk == rowk, 1e9, 0.0).astype(jnp.float32)
        acc = jnp.zeros((rows, _N1), jnp.float32)
        for _ in range(_K):
            m = jnp.min(work, axis=1, keepdims=True)
            eq = work == m
            idx = jnp.min(jnp.where(eq, colk, jnp.int32(2**30)), axis=1,
                          keepdims=True)
            oh = colk == idx
            acc = acc + oh.astype(jnp.float32)
            work = jnp.where(oh, jnp.float32(jnp.inf), work)
        t_ref[:, :_N1] = acc
        t_ref[:, _N1:] = jnp.ones((rows, _N2), jnp.float32)
        yp_ref[...] = jax.lax.dot(acc, y_ref[...], precision=_HP) * (1.0 / _K)

    @pl.when(i >= _N1 // rows)
    def _const():
        col = jax.lax.broadcasted_iota(jnp.int32, (rows, _N), 1)
        t_ref[...] = jnp.where(col < _N1, 1.0, 0.0).astype(jnp.float32)


def _adj_body(a_ref, b_ref, out_ref):
    rows = a_ref.shape[0]
    acc = jnp.zeros((rows, _N), jnp.float32)
    for h in range(_HEADS):
        ah = a_ref[:, h * _EMB:(h + 1) * _EMB]
        bh = b_ref[:, h * _EMB:(h + 1) * _EMB]
        dots = jax.lax.dot_general(ah, bh, (((1,), (1,)), ((), ())),
                                   preferred_element_type=jnp.float32)
        acc = acc + jax.nn.sigmoid(dots)
    out_ref[...] = acc * (1.0 / _HEADS)


def kernel(lm_X, lm_Y, tg_X, tg_Y, lm_delay, tg_delay, emb_W, emb_b, w1, w2):
    feats = jnp.concatenate([
        jnp.concatenate([lm_X, lm_delay[:, None]], axis=1),
        jnp.concatenate([tg_X, tg_delay[:, None]], axis=1),
    ], axis=0)

    pr_rows = 256
    ahat, bhat = pl.pallas_call(
        _prologue_body,
        grid=(_N // pr_rows,),
        in_specs=[
            pl.BlockSpec((pr_rows, _DIM + 1), lambda i: (i, 0)),
            pl.BlockSpec((_DIM + 1, _EMB), lambda i: (0, 0)),
            pl.BlockSpec((1, _EMB), lambda i: (0, 0)),
            pl.BlockSpec((_HEADS, _EMB), lambda i: (0, 0)),
            pl.BlockSpec((_HEADS, _EMB), lambda i: (0, 0)),
        ],
        out_specs=[
            pl.BlockSpec((pr_rows, _HEADS * _EMB), lambda i: (i, 0)),
            pl.BlockSpec((pr_rows, _HEADS * _EMB), lambda i: (i, 0)),
        ],
        out_shape=[
            jax.ShapeDtypeStruct((_N, _HEADS * _EMB), jnp.bfloat16),
            jax.ShapeDtypeStruct((_N, _HEADS * _EMB), jnp.bfloat16),
        ],
    )(feats, emb_W, emb_b.reshape(1, _EMB), w1.T, w2.T)

    k_rows = 128
    nb_lm = _N1 // k_rows
    y_pred, teacher = pl.pallas_call(
        _knn_body,
        grid=(_N // k_rows,),
        in_specs=[
            pl.BlockSpec((k_rows, _DIM), lambda i: (jnp.minimum(i, nb_lm - 1), 0)),
            pl.BlockSpec((_DIM, _N1), lambda i: (0, 0)),
            pl.BlockSpec((_N1, 2), lambda i: (0, 0)),
        ],
        out_specs=[
            pl.BlockSpec((k_rows, 2), lambda i: (jnp.minimum(i, nb_lm - 1), 0)),
            pl.BlockSpec((k_rows, _N), lambda i: (i, 0)),
        ],
        out_shape=[
            jax.ShapeDtypeStruct((_N1, 2), jnp.float32),
            jax.ShapeDtypeStruct((_N, _N), jnp.float32),
        ],
    )(lm_X, lm_X.T, lm_Y)

    a_rows = 128
    adj = pl.pallas_call(
        _adj_body,
        grid=(_N // a_rows,),
        in_specs=[
            pl.BlockSpec((a_rows, _HEADS * _EMB), lambda i: (i, 0)),
            pl.BlockSpec((_N, _HEADS * _EMB), lambda i: (0, 0)),
        ],
        out_specs=pl.BlockSpec((a_rows, _N), lambda i: (i, 0)),
        out_shape=jax.ShapeDtypeStruct((_N, _N), jnp.float32),
    )(ahat, bhat)

    return y_pred, adj, teacher
